# baseline (speedup 1.0000x reference)
import functools

import jax
import jax.numpy as jnp
from jax import lax
from jax.experimental import pallas as pl
from jax.experimental.pallas import tpu as pltpu
from jax.experimental.pallas import tpu_sc as plsc

N = 50000
E = 800000
D_IN = 8
N_PAD = 50176
E_PAD = 802816
N_CHUNKS = 196
CHUNK = 128
ROWS_PER_TILE = N_PAD // 16
RB = 512
GRID = N_PAD // RB

_SC_PARAMS = pltpu.CompilerParams(use_tc_tiling_on_sc=False)


@functools.lru_cache(maxsize=None)
def _mesh():
    return plsc.VectorSubcoreMesh(core_axis_name="c", subcore_axis_name="s")


def _zero_acc(zeros_ref, zb_v, acc_sh, base):
    pltpu.sync_copy(zeros_ref, zb_v)

    def cp(k, carry):
        pltpu.sync_copy(zb_v, acc_sh.at[pl.ds(base + k * CHUNK, CHUNK)])
        return carry

    lax.fori_loop(0, ROWS_PER_TILE // CHUNK, cp, 0)
    rem = ROWS_PER_TILE - (ROWS_PER_TILE // CHUNK) * CHUNK
    if rem:
        pltpu.sync_copy(
            zb_v.at[pl.ds(0, rem)],
            acc_sh.at[pl.ds(base + (ROWS_PER_TILE // CHUNK) * CHUNK, rem)])


@functools.lru_cache(maxsize=None)
def _sc_degree_kernel():
    @functools.partial(
        pl.kernel,
        out_type=jax.ShapeDtypeStruct((2, N_PAD, 8), jnp.float32),
        mesh=_mesh(),
        scratch_types=[
            pltpu.VMEM((1, CHUNK), jnp.int32),
            pltpu.VMEM((1, CHUNK), jnp.int32),
            pltpu.VMEM((CHUNK, 8), jnp.float32),
            pltpu.VMEM((CHUNK, 8), jnp.float32),
            pltpu.SemaphoreType.DMA,
            pltpu.SemaphoreType.DMA,
            pltpu.VMEM_SHARED((N_PAD, 8), jnp.float32),
        ],
        compiler_params=_SC_PARAMS,
    )
    def deg_kernel(dst_ref, ones_ref, zeros_ref, out_ref, dst_a, dst_b,
                   ones_v, zb_v, sem_a, sem_b, acc_sh):
        c = lax.axis_index("c")
        s = lax.axis_index("s")
        base = s * ROWS_PER_TILE
        _zero_acc(zeros_ref, zb_v, acc_sh, base)
        pltpu.sync_copy(ones_ref, ones_v)
        plsc.subcore_barrier()

        pltpu.async_copy(dst_ref.at[c, s, 0], dst_a.at[0], sem_a)
        pltpu.async_copy(dst_ref.at[c, s, 1], dst_b.at[0], sem_b)

        def body(i, carry):
            j0 = 2 * i
            pltpu.make_async_copy(dst_ref.at[c, s, j0], dst_a.at[0],
                                  sem_a).wait()
            pltpu.sync_copy(ones_v, acc_sh.at[dst_a.at[0]], add=True)

            @pl.when(j0 + 2 < N_CHUNKS)
            def _():
                pltpu.async_copy(dst_ref.at[c, s, j0 + 2], dst_a.at[0], sem_a)

            pltpu.make_async_copy(dst_ref.at[c, s, j0 + 1], dst_b.at[0],
                                  sem_b).wait()
            pltpu.sync_copy(ones_v, acc_sh.at[dst_b.at[0]], add=True)

            @pl.when(j0 + 3 < N_CHUNKS)
            def _():
                pltpu.async_copy(dst_ref.at[c, s, j0 + 3], dst_b.at[0], sem_b)

            return carry

        lax.fori_loop(0, N_CHUNKS // 2, body, 0)
        plsc.subcore_barrier()
        pltpu.sync_copy(acc_sh.at[pl.ds(base, ROWS_PER_TILE)],
                        out_ref.at[c, pl.ds(base, ROWS_PER_TILE)])

    return deg_kernel


@functools.lru_cache(maxsize=None)
def _make_sc_propagate(D):
    @functools.partial(
        pl.kernel,
        out_type=jax.ShapeDtypeStruct((2, N_PAD, D), jnp.float32),
        mesh=_mesh(),
        scratch_types=[
            pltpu.VMEM((1, CHUNK), jnp.int32),
            pltpu.VMEM((1, CHUNK), jnp.int32),
            pltpu.VMEM((1, CHUNK), jnp.int32),
            pltpu.VMEM((1, CHUNK), jnp.int32),
            pltpu.VMEM((CHUNK, D), jnp.float32),
            pltpu.VMEM((CHUNK, D), jnp.float32),
            pltpu.SemaphoreType.DMA,
            pltpu.SemaphoreType.DMA,
            pltpu.SemaphoreType.DMA,
            pltpu.SemaphoreType.DMA,
            pltpu.SemaphoreType.DMA,
            pltpu.SemaphoreType.DMA,
            pltpu.VMEM_SHARED((N_PAD, D), jnp.float32),
        ],
        compiler_params=_SC_PARAMS,
    )
    def prop_kernel(table_ref, src_ref, dst_ref, zeros_ref, out_ref,
                    src_a, dst_a, src_b, dst_b, rows_a, rows_b,
                    sem_sa, sem_da, sem_sb, sem_db, sem_ga, sem_gb, acc_sh):
        c = lax.axis_index("c")
        s = lax.axis_index("s")
        base = s * ROWS_PER_TILE
        _zero_acc(zeros_ref, rows_a, acc_sh, base)
        plsc.subcore_barrier()

        pltpu.async_copy(src_ref.at[c, s, 0], src_a.at[0], sem_sa)
        pltpu.async_copy(dst_ref.at[c, s, 0], dst_a.at[0], sem_da)
        pltpu.async_copy(src_ref.at[c, s, 1], src_b.at[0], sem_sb)
        pltpu.async_copy(dst_ref.at[c, s, 1], dst_b.at[0], sem_db)

        def body(i, carry):
            j0 = 2 * i
            pltpu.make_async_copy(src_ref.at[c, s, j0], src_a.at[0],
                                  sem_sa).wait()
            cp_a = pltpu.async_copy(table_ref.at[src_a.at[0]], rows_a, sem_ga)
            pltpu.make_async_copy(src_ref.at[c, s, j0 + 1], src_b.at[0],
                                  sem_sb).wait()
            cp_b = pltpu.async_copy(table_ref.at[src_b.at[0]], rows_b, sem_gb)

            pltpu.make_async_copy(dst_ref.at[c, s, j0], dst_a.at[0],
                                  sem_da).wait()
            cp_a.wait()
            pltpu.sync_copy(rows_a, acc_sh.at[dst_a.at[0]], add=True)

            @pl.when(j0 + 2 < N_CHUNKS)
            def _():
                pltpu.async_copy(src_ref.at[c, s, j0 + 2], src_a.at[0], sem_sa)
                pltpu.async_copy(dst_ref.at[c, s, j0 + 2], dst_a.at[0], sem_da)

            pltpu.make_async_copy(dst_ref.at[c, s, j0 + 1], dst_b.at[0],
                                  sem_db).wait()
            cp_b.wait()
            pltpu.sync_copy(rows_b, acc_sh.at[dst_b.at[0]], add=True)

            @pl.when(j0 + 3 < N_CHUNKS)
            def _():
                pltpu.async_copy(src_ref.at[c, s, j0 + 3], src_b.at[0], sem_sb)
                pltpu.async_copy(dst_ref.at[c, s, j0 + 3], dst_b.at[0], sem_db)

            return carry

        lax.fori_loop(0, N_CHUNKS // 2, body, 0)
        plsc.subcore_barrier()
        pltpu.sync_copy(acc_sh.at[pl.ds(base, ROWS_PER_TILE)],
                        out_ref.at[c, pl.ds(base, ROWS_PER_TILE)])

    return prop_kernel


def _tc_k1(degA, degB, x_pad):
    def body(dA, dB, xb, dinv_ref, t1_ref):
        deg = dA[:, 0:1] + dB[:, 0:1] + 1.0
        dinv = lax.rsqrt(deg)
        dinv_ref[...] = dinv
        t1_ref[...] = xb[...] * dinv

    return pl.pallas_call(
        body,
        grid=(GRID,),
        in_specs=[
            pl.BlockSpec((RB, 8), lambda i: (i, 0)),
            pl.BlockSpec((RB, 8), lambda i: (i, 0)),
            pl.BlockSpec((RB, 8), lambda i: (i, 0)),
        ],
        out_specs=[
            pl.BlockSpec((RB, 1), lambda i: (i, 0)),
            pl.BlockSpec((RB, 8), lambda i: (i, 0)),
        ],
        out_shape=[
            jax.ShapeDtypeStruct((N_PAD, 1), jnp.float32),
            jax.ShapeDtypeStruct((N_PAD, 8), jnp.float32),
        ],
    )(degA, degB, x_pad)


def _tc_k2(u1a, u1b, t1, dinv, W1, b1, W2a, W2b):
    def body(ua, ub, t1b, dv, w1, bias1, w2a, w2b, t2a_ref, t2b_ref):
        i = pl.program_id(0)
        sh = dv[...] * (ua[...] + ub[...] + t1b[...])
        h1 = jnp.maximum(
            jnp.dot(sh, w1[...], preferred_element_type=jnp.float32) + bias1[...],
            0.0)
        rows = i * RB + lax.broadcasted_iota(jnp.int32, (RB, 1), 0)
        h1 = jnp.where(rows < N, h1, 0.0)
        t2a_ref[...] = dv[...] * jnp.dot(h1, w2a[...],
                                         preferred_element_type=jnp.float32)
        t2b_ref[...] = dv[...] * jnp.dot(h1, w2b[...],
                                         preferred_element_type=jnp.float32)

    return pl.pallas_call(
        body,
        grid=(GRID,),
        in_specs=[
            pl.BlockSpec((RB, 8), lambda i: (i, 0)),
            pl.BlockSpec((RB, 8), lambda i: (i, 0)),
            pl.BlockSpec((RB, 8), lambda i: (i, 0)),
            pl.BlockSpec((RB, 1), lambda i: (i, 0)),
            pl.BlockSpec((8, 128), lambda i: (0, 0)),
            pl.BlockSpec((1, 128), lambda i: (0, 0)),
            pl.BlockSpec((128, 32), lambda i: (0, 0)),
            pl.BlockSpec((128, 32), lambda i: (0, 0)),
        ],
        out_specs=[
            pl.BlockSpec((RB, 32), lambda i: (i, 0)),
            pl.BlockSpec((RB, 32), lambda i: (i, 0)),
        ],
        out_shape=[
            jax.ShapeDtypeStruct((N_PAD, 32), jnp.float32),
            jax.ShapeDtypeStruct((N_PAD, 32), jnp.float32),
        ],
    )(u1a, u1b, t1, dinv, W1, b1, W2a, W2b)


def _tc_k3(u2a0, u2a1, u2b0, u2b1, t2a, t2b, dinv, b2, W3):
    def body(a0, a1, b0, b1r, ta, tb, dv, bias2, w3, t3_ref):
        i = pl.program_id(0)
        ha = dv[...] * (a0[...] + a1[...] + ta[...])
        hb = dv[...] * (b0[...] + b1r[...] + tb[...])
        h2 = jnp.maximum(jnp.concatenate([ha, hb], axis=1) + bias2[...], 0.0)
        rows = i * RB + lax.broadcasted_iota(jnp.int32, (RB, 1), 0)
        h2 = jnp.where(rows < N, h2, 0.0)
        t3_ref[...] = dv[...] * jnp.dot(h2, w3[...],
                                        preferred_element_type=jnp.float32)

    return pl.pallas_call(
        body,
        grid=(GRID,),
        in_specs=[
            pl.BlockSpec((RB, 32), lambda i: (i, 0)),
            pl.BlockSpec((RB, 32), lambda i: (i, 0)),
            pl.BlockSpec((RB, 32), lambda i: (i, 0)),
            pl.BlockSpec((RB, 32), lambda i: (i, 0)),
            pl.BlockSpec((RB, 32), lambda i: (i, 0)),
            pl.BlockSpec((RB, 32), lambda i: (i, 0)),
            pl.BlockSpec((RB, 1), lambda i: (i, 0)),
            pl.BlockSpec((1, 64), lambda i: (0, 0)),
            pl.BlockSpec((64, 32), lambda i: (0, 0)),
        ],
        out_specs=pl.BlockSpec((RB, 32), lambda i: (i, 0)),
        out_shape=jax.ShapeDtypeStruct((N_PAD, 32), jnp.float32),
    )(u2a0, u2a1, u2b0, u2b1, t2a, t2b, dinv, b2, W3)


def _tc_k4(u3a, u3b, t3, dinv, b3, Wp1, bp1, Wp2, bp2):
    def body(ua, ub, tb, dv, bias3, wp1, biasp1, wp2, biasp2, out_ref, acc):
        i = pl.program_id(0)
        h3 = jnp.maximum(dv[...] * (ua[...] + ub[...] + tb[...]) + bias3[...],
                         0.0)
        rows = i * RB + lax.broadcasted_iota(jnp.int32, (RB, 1), 0)
        h3 = jnp.where(rows < N, h3, 0.0)
        part = jnp.sum(h3, axis=0, keepdims=True)

        @pl.when(i == 0)
        def _():
            acc[...] = part

        @pl.when(i > 0)
        def _():
            acc[...] = acc[...] + part

        @pl.when(i == GRID - 1)
        def _():
            g = acc[...] * (1.0 / N)
            p = jnp.maximum(
                jnp.dot(g, wp1[...], preferred_element_type=jnp.float32)
                + biasp1[...], 0.0)
            out_ref[...] = (jnp.dot(p, wp2[...],
                                    preferred_element_type=jnp.float32)
                            + biasp2[...])

    return pl.pallas_call(
        body,
        grid=(GRID,),
        in_specs=[
            pl.BlockSpec((RB, 32), lambda i: (i, 0)),
            pl.BlockSpec((RB, 32), lambda i: (i, 0)),
            pl.BlockSpec((RB, 32), lambda i: (i, 0)),
            pl.BlockSpec((RB, 1), lambda i: (i, 0)),
            pl.BlockSpec((1, 32), lambda i: (0, 0)),
            pl.BlockSpec((32, 16), lambda i: (0, 0)),
            pl.BlockSpec((1, 16), lambda i: (0, 0)),
            pl.BlockSpec((16, 1), lambda i: (0, 0)),
            pl.BlockSpec((1, 1), lambda i: (0, 0)),
        ],
        out_specs=pl.BlockSpec((1, 1), lambda i: (0, 0)),
        out_shape=jax.ShapeDtypeStruct((1, 1), jnp.float32),
        scratch_shapes=[pltpu.VMEM((1, 32), jnp.float32)],
    )(u3a, u3b, t3, dinv, b3, Wp1, bp1, Wp2, bp2)


def kernel(x, edge_index, W1, b1, W2, b2, W3, b3, Wp1, bp1, Wp2, bp2):
    pad_cols = jnp.full((2, E_PAD - E), N, jnp.int32)
    ei = jnp.concatenate([edge_index, pad_cols], axis=1)
    src_hbm = ei[0].reshape(2, 16, N_CHUNKS, CHUNK)
    dst_hbm = ei[1].reshape(2, 16, N_CHUNKS, CHUNK)

    x_pad = jnp.zeros((N_PAD, D_IN), jnp.float32).at[:N].set(x)
    ones8 = jnp.ones((CHUNK, 8), jnp.float32)
    zeros8 = jnp.zeros((CHUNK, 8), jnp.float32)
    zeros32 = jnp.zeros((CHUNK, 32), jnp.float32)

    deg = _sc_degree_kernel()(dst_hbm, ones8, zeros8)
    dinv, t1 = _tc_k1(deg[0], deg[1], x_pad)

    u1 = _make_sc_propagate(8)(t1, src_hbm, dst_hbm, zeros8)
    t2a, t2b = _tc_k2(u1[0], u1[1], t1, dinv,
                      W1, b1.reshape(1, 128), W2[:, :32], W2[:, 32:])

    u2a = _make_sc_propagate(32)(t2a, src_hbm, dst_hbm, zeros32)
    u2b = _make_sc_propagate(32)(t2b, src_hbm, dst_hbm, zeros32)
    t3 = _tc_k3(u2a[0], u2a[1], u2b[0], u2b[1], t2a, t2b, dinv,
                b2.reshape(1, 64), W3)

    u3 = _make_sc_propagate(32)(t3, src_hbm, dst_hbm, zeros32)
    out = _tc_k4(u3[0], u3[1], t3, dinv, b3.reshape(1, 32),
                 Wp1, bp1.reshape(1, 16), Wp2, bp2.reshape(1, 1))
    return out.reshape(1)


# speedup vs baseline: 1.2999x; 1.2999x over previous
import functools

import jax
import jax.numpy as jnp
from jax import lax
from jax.experimental import pallas as pl
from jax.experimental.pallas import tpu as pltpu
from jax.experimental.pallas import tpu_sc as plsc

N = 50000
E = 800000
D_IN = 8
N_PAD = 50176
E_PAD = 802816
N_CHUNKS = 196
CHUNK = 128
ROWS_PER_TILE = N_PAD // 16
RB = 3584
GRID = N_PAD // RB

_SC_PARAMS = pltpu.CompilerParams(use_tc_tiling_on_sc=False)


@functools.lru_cache(maxsize=None)
def _mesh():
    return plsc.VectorSubcoreMesh(core_axis_name="c", subcore_axis_name="s")


def _zero_acc(zeros_ref, zb_v, acc_sh, base):
    pltpu.sync_copy(zeros_ref, zb_v)

    def cp(k, carry):
        pltpu.sync_copy(zb_v, acc_sh.at[pl.ds(base + k * CHUNK, CHUNK)])
        return carry

    lax.fori_loop(0, ROWS_PER_TILE // CHUNK, cp, 0)
    rem = ROWS_PER_TILE - (ROWS_PER_TILE // CHUNK) * CHUNK
    if rem:
        pltpu.sync_copy(
            zb_v.at[pl.ds(0, rem)],
            acc_sh.at[pl.ds(base + (ROWS_PER_TILE // CHUNK) * CHUNK, rem)])


@functools.lru_cache(maxsize=None)
def _sc_degree_kernel():
    @functools.partial(
        pl.kernel,
        out_type=jax.ShapeDtypeStruct((2, N_PAD, 8), jnp.float32),
        mesh=_mesh(),
        scratch_types=[
            pltpu.VMEM((1, CHUNK), jnp.int32),
            pltpu.VMEM((1, CHUNK), jnp.int32),
            pltpu.VMEM((CHUNK, 8), jnp.float32),
            pltpu.VMEM((CHUNK, 8), jnp.float32),
            pltpu.SemaphoreType.DMA,
            pltpu.SemaphoreType.DMA,
            pltpu.VMEM_SHARED((N_PAD, 8), jnp.float32),
        ],
        compiler_params=_SC_PARAMS,
    )
    def deg_kernel(dst_ref, ones_ref, zeros_ref, out_ref, dst_a, dst_b,
                   ones_v, zb_v, sem_a, sem_b, acc_sh):
        c = lax.axis_index("c")
        s = lax.axis_index("s")
        base = s * ROWS_PER_TILE
        _zero_acc(zeros_ref, zb_v, acc_sh, base)
        pltpu.sync_copy(ones_ref, ones_v)
        plsc.subcore_barrier()

        pltpu.async_copy(dst_ref.at[c, s, 0], dst_a.at[0], sem_a)
        pltpu.async_copy(dst_ref.at[c, s, 1], dst_b.at[0], sem_b)

        def body(i, carry):
            j0 = 2 * i
            pltpu.make_async_copy(dst_ref.at[c, s, j0], dst_a.at[0],
                                  sem_a).wait()
            pltpu.sync_copy(ones_v, acc_sh.at[dst_a.at[0]], add=True)

            @pl.when(j0 + 2 < N_CHUNKS)
            def _():
                pltpu.async_copy(dst_ref.at[c, s, j0 + 2], dst_a.at[0], sem_a)

            pltpu.make_async_copy(dst_ref.at[c, s, j0 + 1], dst_b.at[0],
                                  sem_b).wait()
            pltpu.sync_copy(ones_v, acc_sh.at[dst_b.at[0]], add=True)

            @pl.when(j0 + 3 < N_CHUNKS)
            def _():
                pltpu.async_copy(dst_ref.at[c, s, j0 + 3], dst_b.at[0], sem_b)

            return carry

        lax.fori_loop(0, N_CHUNKS // 2, body, 0)
        plsc.subcore_barrier()
        pltpu.sync_copy(acc_sh.at[pl.ds(base, ROWS_PER_TILE)],
                        out_ref.at[c, pl.ds(base, ROWS_PER_TILE)])

    return deg_kernel


@functools.lru_cache(maxsize=None)
def _make_sc_propagate(D):
    @functools.partial(
        pl.kernel,
        out_type=jax.ShapeDtypeStruct((2, N_PAD, D), jnp.float32),
        mesh=_mesh(),
        scratch_types=[
            pltpu.VMEM((1, CHUNK), jnp.int32),
            pltpu.VMEM((1, CHUNK), jnp.int32),
            pltpu.VMEM((1, CHUNK), jnp.int32),
            pltpu.VMEM((1, CHUNK), jnp.int32),
            pltpu.VMEM((CHUNK, D), jnp.float32),
            pltpu.VMEM((CHUNK, D), jnp.float32),
            pltpu.SemaphoreType.DMA,
            pltpu.SemaphoreType.DMA,
            pltpu.SemaphoreType.DMA,
            pltpu.SemaphoreType.DMA,
            pltpu.SemaphoreType.DMA,
            pltpu.SemaphoreType.DMA,
            pltpu.VMEM_SHARED((N_PAD, D), jnp.float32),
        ],
        compiler_params=_SC_PARAMS,
    )
    def prop_kernel(table_ref, src_ref, dst_ref, zeros_ref, out_ref,
                    src_a, dst_a, src_b, dst_b, rows_a, rows_b,
                    sem_sa, sem_da, sem_sb, sem_db, sem_ga, sem_gb, acc_sh):
        c = lax.axis_index("c")
        s = lax.axis_index("s")
        base = s * ROWS_PER_TILE
        _zero_acc(zeros_ref, rows_a, acc_sh, base)
        plsc.subcore_barrier()

        pltpu.async_copy(src_ref.at[c, s, 0], src_a.at[0], sem_sa)
        pltpu.async_copy(dst_ref.at[c, s, 0], dst_a.at[0], sem_da)
        pltpu.async_copy(src_ref.at[c, s, 1], src_b.at[0], sem_sb)
        pltpu.async_copy(dst_ref.at[c, s, 1], dst_b.at[0], sem_db)

        def body(i, carry):
            j0 = 2 * i
            pltpu.make_async_copy(src_ref.at[c, s, j0], src_a.at[0],
                                  sem_sa).wait()
            cp_a = pltpu.async_copy(table_ref.at[src_a.at[0]], rows_a, sem_ga)
            pltpu.make_async_copy(src_ref.at[c, s, j0 + 1], src_b.at[0],
                                  sem_sb).wait()
            cp_b = pltpu.async_copy(table_ref.at[src_b.at[0]], rows_b, sem_gb)

            pltpu.make_async_copy(dst_ref.at[c, s, j0], dst_a.at[0],
                                  sem_da).wait()
            cp_a.wait()
            pltpu.sync_copy(rows_a, acc_sh.at[dst_a.at[0]], add=True)

            @pl.when(j0 + 2 < N_CHUNKS)
            def _():
                pltpu.async_copy(src_ref.at[c, s, j0 + 2], src_a.at[0], sem_sa)
                pltpu.async_copy(dst_ref.at[c, s, j0 + 2], dst_a.at[0], sem_da)

            pltpu.make_async_copy(dst_ref.at[c, s, j0 + 1], dst_b.at[0],
                                  sem_db).wait()
            cp_b.wait()
            pltpu.sync_copy(rows_b, acc_sh.at[dst_b.at[0]], add=True)

            @pl.when(j0 + 3 < N_CHUNKS)
            def _():
                pltpu.async_copy(src_ref.at[c, s, j0 + 3], src_b.at[0], sem_sb)
                pltpu.async_copy(dst_ref.at[c, s, j0 + 3], dst_b.at[0], sem_db)

            return carry

        lax.fori_loop(0, N_CHUNKS // 2, body, 0)
        plsc.subcore_barrier()
        pltpu.sync_copy(acc_sh.at[pl.ds(base, ROWS_PER_TILE)],
                        out_ref.at[c, pl.ds(base, ROWS_PER_TILE)])

    return prop_kernel


def _sp(shape, imap):
    return pl.BlockSpec(shape, imap)


def _tc_k1(deg3, x):
    def body(dA, dB, xb, dinv_ref, t1_ref):
        i = pl.program_id(0)
        deg = dA[0][:, 0:1] + dB[0][:, 0:1] + 1.0
        dinv = lax.rsqrt(deg)
        rows = i * RB + lax.broadcasted_iota(jnp.int32, (RB, 1), 0)
        dinv_ref[...] = dinv
        t1_ref[...] = jnp.where(rows < N, xb[...] * dinv, 0.0)

    return pl.pallas_call(
        body,
        grid=(GRID,),
        in_specs=[
            _sp((1, RB, 8), lambda i: (0, i, 0)),
            _sp((1, RB, 8), lambda i: (1, i, 0)),
            _sp((RB, 8), lambda i: (i, 0)),
        ],
        out_specs=[
            _sp((RB, 1), lambda i: (i, 0)),
            _sp((RB, 8), lambda i: (i, 0)),
        ],
        out_shape=[
            jax.ShapeDtypeStruct((N_PAD, 1), jnp.float32),
            jax.ShapeDtypeStruct((N_PAD, 8), jnp.float32),
        ],
    )(deg3, deg3, x)


def _tc_k2(u1, t1, dinv, W1, b1, W2a, W2b):
    def body(ua, ub, t1b, dv, w1, bias1, w2a, w2b, t2a_ref, t2b_ref):
        i = pl.program_id(0)
        sh = dv[...] * (ua[0] + ub[0] + t1b[...])
        h1 = jnp.maximum(
            jnp.dot(sh, w1[...], preferred_element_type=jnp.float32) + bias1[...],
            0.0)
        rows = i * RB + lax.broadcasted_iota(jnp.int32, (RB, 1), 0)
        h1 = jnp.where(rows < N, h1, 0.0)
        t2a_ref[...] = dv[...] * jnp.dot(h1, w2a[...],
                                         preferred_element_type=jnp.float32)
        t2b_ref[...] = dv[...] * jnp.dot(h1, w2b[...],
                                         preferred_element_type=jnp.float32)

    return pl.pallas_call(
        body,
        grid=(GRID,),
        in_specs=[
            _sp((1, RB, 8), lambda i: (0, i, 0)),
            _sp((1, RB, 8), lambda i: (1, i, 0)),
            _sp((RB, 8), lambda i: (i, 0)),
            _sp((RB, 1), lambda i: (i, 0)),
            _sp((8, 128), lambda i: (0, 0)),
            _sp((1, 128), lambda i: (0, 0)),
            _sp((128, 32), lambda i: (0, 0)),
            _sp((128, 32), lambda i: (0, 0)),
        ],
        out_specs=[
            _sp((RB, 32), lambda i: (i, 0)),
            _sp((RB, 32), lambda i: (i, 0)),
        ],
        out_shape=[
            jax.ShapeDtypeStruct((N_PAD, 32), jnp.float32),
            jax.ShapeDtypeStruct((N_PAD, 32), jnp.float32),
        ],
    )(u1, u1, t1, dinv, W1, b1, W2a, W2b)


def _tc_k3(u2a, u2b, t2a, t2b, dinv, b2, W3):
    def body(a0, a1, b0, b1r, ta, tb, dv, bias2, w3, t3_ref):
        i = pl.program_id(0)
        ha = dv[...] * (a0[0] + a1[0] + ta[...])
        hb = dv[...] * (b0[0] + b1r[0] + tb[...])
        h2 = jnp.maximum(jnp.concatenate([ha, hb], axis=1) + bias2[...], 0.0)
        rows = i * RB + lax.broadcasted_iota(jnp.int32, (RB, 1), 0)
        h2 = jnp.where(rows < N, h2, 0.0)
        t3_ref[...] = dv[...] * jnp.dot(h2, w3[...],
                                        preferred_element_type=jnp.float32)

    return pl.pallas_call(
        body,
        grid=(GRID,),
        in_specs=[
            _sp((1, RB, 32), lambda i: (0, i, 0)),
            _sp((1, RB, 32), lambda i: (1, i, 0)),
            _sp((1, RB, 32), lambda i: (0, i, 0)),
            _sp((1, RB, 32), lambda i: (1, i, 0)),
            _sp((RB, 32), lambda i: (i, 0)),
            _sp((RB, 32), lambda i: (i, 0)),
            _sp((RB, 1), lambda i: (i, 0)),
            _sp((1, 64), lambda i: (0, 0)),
            _sp((64, 32), lambda i: (0, 0)),
        ],
        out_specs=_sp((RB, 32), lambda i: (i, 0)),
        out_shape=jax.ShapeDtypeStruct((N_PAD, 32), jnp.float32),
    )(u2a, u2a, u2b, u2b, t2a, t2b, dinv, b2, W3)


def _tc_k4(u3, t3, dinv, b3, Wp1, bp1, Wp2, bp2):
    def body(ua, ub, tb, dv, bias3, wp1, biasp1, wp2, biasp2, out_ref, acc):
        i = pl.program_id(0)
        h3 = jnp.maximum(dv[...] * (ua[0] + ub[0] + tb[...]) + bias3[...],
                         0.0)
        rows = i * RB + lax.broadcasted_iota(jnp.int32, (RB, 1), 0)
        h3 = jnp.where(rows < N, h3, 0.0)
        part = jnp.sum(h3, axis=0, keepdims=True)

        @pl.when(i == 0)
        def _():
            acc[...] = part

        @pl.when(i > 0)
        def _():
            acc[...] = acc[...] + part

        @pl.when(i == GRID - 1)
        def _():
            g = acc[...] * (1.0 / N)
            p = jnp.maximum(
                jnp.dot(g, wp1[...], preferred_element_type=jnp.float32)
                + biasp1[...], 0.0)
            out_ref[...] = (jnp.dot(p, wp2[...],
                                    preferred_element_type=jnp.float32)
                            + biasp2[...])

    return pl.pallas_call(
        body,
        grid=(GRID,),
        in_specs=[
            _sp((1, RB, 32), lambda i: (0, i, 0)),
            _sp((1, RB, 32), lambda i: (1, i, 0)),
            _sp((RB, 32), lambda i: (i, 0)),
            _sp((RB, 1), lambda i: (i, 0)),
            _sp((1, 32), lambda i: (0, 0)),
            _sp((32, 16), lambda i: (0, 0)),
            _sp((1, 16), lambda i: (0, 0)),
            _sp((16, 1), lambda i: (0, 0)),
            _sp((1, 1), lambda i: (0, 0)),
        ],
        out_specs=_sp((1, 1), lambda i: (0, 0)),
        out_shape=jax.ShapeDtypeStruct((1, 1), jnp.float32),
        scratch_shapes=[pltpu.VMEM((1, 32), jnp.float32)],
    )(u3, u3, t3, dinv, b3, Wp1, bp1, Wp2, bp2)


def kernel(x, edge_index, W1, b1, W2, b2, W3, b3, Wp1, bp1, Wp2, bp2):
    pad_cols = jnp.full((2, E_PAD - E), N, jnp.int32)
    ei = jnp.concatenate([edge_index, pad_cols], axis=1)
    src_hbm = ei[0].reshape(2, 16, N_CHUNKS, CHUNK)
    dst_hbm = ei[1].reshape(2, 16, N_CHUNKS, CHUNK)

    ones8 = jnp.ones((CHUNK, 8), jnp.float32)
    zeros8 = jnp.zeros((CHUNK, 8), jnp.float32)
    zeros32 = jnp.zeros((CHUNK, 32), jnp.float32)

    deg = _sc_degree_kernel()(dst_hbm, ones8, zeros8)
    dinv, t1 = _tc_k1(deg, x)

    u1 = _make_sc_propagate(8)(t1, src_hbm, dst_hbm, zeros8)
    t2a, t2b = _tc_k2(u1, t1, dinv,
                      W1, b1.reshape(1, 128), W2[:, :32], W2[:, 32:])

    u2a = _make_sc_propagate(32)(t2a, src_hbm, dst_hbm, zeros32)
    u2b = _make_sc_propagate(32)(t2b, src_hbm, dst_hbm, zeros32)
    t3 = _tc_k3(u2a, u2b, t2a, t2b, dinv, b2.reshape(1, 64), W3)

    u3 = _make_sc_propagate(32)(t3, src_hbm, dst_hbm, zeros32)
    out = _tc_k4(u3, t3, dinv, b3.reshape(1, 32),
                 Wp1, bp1.reshape(1, 16), Wp2, bp2.reshape(1, 1))
    return out.reshape(1)


# speedup vs baseline: 1.5575x; 1.1981x over previous
import functools

import jax
import jax.numpy as jnp
from jax import lax
from jax.experimental import pallas as pl
from jax.experimental.pallas import tpu as pltpu
from jax.experimental.pallas import tpu_sc as plsc

N = 50000
E = 800000
D_IN = 8
N_PAD = 50176
E_PAD = 802816
N_CHUNKS = 196
CHUNK = 128
ROWS_PER_TILE = N_PAD // 16
RB = 3584
GRID = N_PAD // RB

_SC_PARAMS = pltpu.CompilerParams(use_tc_tiling_on_sc=False)


@functools.lru_cache(maxsize=None)
def _mesh():
    return plsc.VectorSubcoreMesh(core_axis_name="c", subcore_axis_name="s")


def _zero_acc(zeros_ref, zb_v, acc_sh, base):
    pltpu.sync_copy(zeros_ref, zb_v)

    def cp(k, carry):
        pltpu.sync_copy(zb_v, acc_sh.at[pl.ds(base + k * CHUNK, CHUNK)])
        return carry

    lax.fori_loop(0, ROWS_PER_TILE // CHUNK, cp, 0)
    rem = ROWS_PER_TILE - (ROWS_PER_TILE // CHUNK) * CHUNK
    if rem:
        pltpu.sync_copy(
            zb_v.at[pl.ds(0, rem)],
            acc_sh.at[pl.ds(base + (ROWS_PER_TILE // CHUNK) * CHUNK, rem)])


@functools.lru_cache(maxsize=None)
def _sc_degree_kernel():
    @functools.partial(
        pl.kernel,
        out_type=jax.ShapeDtypeStruct((2, N_PAD, 8), jnp.float32),
        mesh=_mesh(),
        scratch_types=[
            pltpu.VMEM((1, CHUNK), jnp.int32),
            pltpu.VMEM((1, CHUNK), jnp.int32),
            pltpu.VMEM((CHUNK, 8), jnp.float32),
            pltpu.VMEM((CHUNK, 8), jnp.float32),
            pltpu.SemaphoreType.DMA,
            pltpu.SemaphoreType.DMA,
            pltpu.VMEM_SHARED((N_PAD, 8), jnp.float32),
        ],
        compiler_params=_SC_PARAMS,
    )
    def deg_kernel(dst_ref, ones_ref, zeros_ref, out_ref, dst_a, dst_b,
                   ones_v, zb_v, sem_a, sem_b, acc_sh):
        c = lax.axis_index("c")
        s = lax.axis_index("s")
        base = s * ROWS_PER_TILE
        _zero_acc(zeros_ref, zb_v, acc_sh, base)
        pltpu.sync_copy(ones_ref, ones_v)
        plsc.subcore_barrier()

        pltpu.async_copy(dst_ref.at[c, s, 0], dst_a.at[0], sem_a)
        pltpu.async_copy(dst_ref.at[c, s, 1], dst_b.at[0], sem_b)

        def body(i, carry):
            j0 = 2 * i
            pltpu.make_async_copy(dst_ref.at[c, s, j0], dst_a.at[0],
                                  sem_a).wait()
            pltpu.sync_copy(ones_v, acc_sh.at[dst_a.at[0]], add=True)

            @pl.when(j0 + 2 < N_CHUNKS)
            def _():
                pltpu.async_copy(dst_ref.at[c, s, j0 + 2], dst_a.at[0], sem_a)

            pltpu.make_async_copy(dst_ref.at[c, s, j0 + 1], dst_b.at[0],
                                  sem_b).wait()
            pltpu.sync_copy(ones_v, acc_sh.at[dst_b.at[0]], add=True)

            @pl.when(j0 + 3 < N_CHUNKS)
            def _():
                pltpu.async_copy(dst_ref.at[c, s, j0 + 3], dst_b.at[0], sem_b)

            return carry

        lax.fori_loop(0, N_CHUNKS // 2, body, 0)
        plsc.subcore_barrier()
        pltpu.sync_copy(acc_sh.at[pl.ds(base, ROWS_PER_TILE)],
                        out_ref.at[c, pl.ds(base, ROWS_PER_TILE)])

    return deg_kernel


@functools.lru_cache(maxsize=None)
def _make_sc_propagate(D, dtype=jnp.float32):
    @functools.partial(
        pl.kernel,
        out_type=jax.ShapeDtypeStruct((2, N_PAD, D), dtype),
        mesh=_mesh(),
        scratch_types=[
            pltpu.VMEM((1, CHUNK), jnp.int32),
            pltpu.VMEM((1, CHUNK), jnp.int32),
            pltpu.VMEM((1, CHUNK), jnp.int32),
            pltpu.VMEM((1, CHUNK), jnp.int32),
            pltpu.VMEM((CHUNK, D), dtype),
            pltpu.VMEM((CHUNK, D), dtype),
            pltpu.SemaphoreType.DMA,
            pltpu.SemaphoreType.DMA,
            pltpu.SemaphoreType.DMA,
            pltpu.SemaphoreType.DMA,
            pltpu.SemaphoreType.DMA,
            pltpu.SemaphoreType.DMA,
            pltpu.VMEM_SHARED((N_PAD, D), dtype),
        ],
        compiler_params=_SC_PARAMS,
    )
    def prop_kernel(table_ref, src_ref, dst_ref, zeros_ref, out_ref,
                    src_a, dst_a, src_b, dst_b, rows_a, rows_b,
                    sem_sa, sem_da, sem_sb, sem_db, sem_ga, sem_gb, acc_sh):
        c = lax.axis_index("c")
        s = lax.axis_index("s")
        base = s * ROWS_PER_TILE
        _zero_acc(zeros_ref, rows_a, acc_sh, base)
        plsc.subcore_barrier()

        pltpu.async_copy(src_ref.at[c, s, 0], src_a.at[0], sem_sa)
        pltpu.async_copy(dst_ref.at[c, s, 0], dst_a.at[0], sem_da)
        pltpu.async_copy(src_ref.at[c, s, 1], src_b.at[0], sem_sb)
        pltpu.async_copy(dst_ref.at[c, s, 1], dst_b.at[0], sem_db)

        def body(i, carry):
            j0 = 2 * i
            pltpu.make_async_copy(src_ref.at[c, s, j0], src_a.at[0],
                                  sem_sa).wait()
            cp_a = pltpu.async_copy(table_ref.at[src_a.at[0]], rows_a, sem_ga)
            pltpu.make_async_copy(src_ref.at[c, s, j0 + 1], src_b.at[0],
                                  sem_sb).wait()
            cp_b = pltpu.async_copy(table_ref.at[src_b.at[0]], rows_b, sem_gb)

            pltpu.make_async_copy(dst_ref.at[c, s, j0], dst_a.at[0],
                                  sem_da).wait()
            cp_a.wait()
            pltpu.sync_copy(rows_a, acc_sh.at[dst_a.at[0]], add=True)

            @pl.when(j0 + 2 < N_CHUNKS)
            def _():
                pltpu.async_copy(src_ref.at[c, s, j0 + 2], src_a.at[0], sem_sa)
                pltpu.async_copy(dst_ref.at[c, s, j0 + 2], dst_a.at[0], sem_da)

            pltpu.make_async_copy(dst_ref.at[c, s, j0 + 1], dst_b.at[0],
                                  sem_db).wait()
            cp_b.wait()
            pltpu.sync_copy(rows_b, acc_sh.at[dst_b.at[0]], add=True)

            @pl.when(j0 + 3 < N_CHUNKS)
            def _():
                pltpu.async_copy(src_ref.at[c, s, j0 + 3], src_b.at[0], sem_sb)
                pltpu.async_copy(dst_ref.at[c, s, j0 + 3], dst_b.at[0], sem_db)

            return carry

        lax.fori_loop(0, N_CHUNKS // 2, body, 0)
        plsc.subcore_barrier()
        pltpu.sync_copy(acc_sh.at[pl.ds(base, ROWS_PER_TILE)],
                        out_ref.at[c, pl.ds(base, ROWS_PER_TILE)])

    return prop_kernel


def _sp(shape, imap):
    return pl.BlockSpec(shape, imap)


def _tc_k1(deg3, x):
    def body(dA, dB, xb, dinv_ref, t1_ref):
        i = pl.program_id(0)
        deg = dA[0][:, 0:1] + dB[0][:, 0:1] + 1.0
        dinv = lax.rsqrt(deg)
        rows = i * RB + lax.broadcasted_iota(jnp.int32, (RB, 1), 0)
        dinv_ref[...] = dinv
        t1_ref[...] = jnp.where(rows < N, xb[...] * dinv, 0.0)

    return pl.pallas_call(
        body,
        grid=(GRID,),
        in_specs=[
            _sp((1, RB, 8), lambda i: (0, i, 0)),
            _sp((1, RB, 8), lambda i: (1, i, 0)),
            _sp((RB, 8), lambda i: (i, 0)),
        ],
        out_specs=[
            _sp((RB, 1), lambda i: (i, 0)),
            _sp((RB, 8), lambda i: (i, 0)),
        ],
        out_shape=[
            jax.ShapeDtypeStruct((N_PAD, 1), jnp.float32),
            jax.ShapeDtypeStruct((N_PAD, 8), jnp.float32),
        ],
    )(deg3, deg3, x)


def _tc_k2(u1, t1, dinv, W1, b1, W2):
    def body(ua, ub, t1b, dv, w1, bias1, w2, t2_ref):
        i = pl.program_id(0)
        sh = dv[...] * (ua[0] + ub[0] + t1b[...])
        h1 = jnp.maximum(
            jnp.dot(sh, w1[...], preferred_element_type=jnp.float32) + bias1[...],
            0.0)
        rows = i * RB + lax.broadcasted_iota(jnp.int32, (RB, 1), 0)
        h1 = jnp.where(rows < N, h1, 0.0)
        t2 = dv[...] * jnp.dot(h1, w2[...], preferred_element_type=jnp.float32)
        t2_ref[...] = t2.astype(jnp.bfloat16)

    return pl.pallas_call(
        body,
        grid=(GRID,),
        in_specs=[
            _sp((1, RB, 8), lambda i: (0, i, 0)),
            _sp((1, RB, 8), lambda i: (1, i, 0)),
            _sp((RB, 8), lambda i: (i, 0)),
            _sp((RB, 1), lambda i: (i, 0)),
            _sp((8, 128), lambda i: (0, 0)),
            _sp((1, 128), lambda i: (0, 0)),
            _sp((128, 64), lambda i: (0, 0)),
        ],
        out_specs=_sp((RB, 64), lambda i: (i, 0)),
        out_shape=jax.ShapeDtypeStruct((N_PAD, 64), jnp.bfloat16),
    )(u1, u1, t1, dinv, W1, b1, W2)


def _tc_k3(u2, t2, dinv, b2, W3):
    def body(a0, a1, ta, dv, bias2, w3, t3_ref):
        i = pl.program_id(0)
        u2f = a0[0].astype(jnp.float32) + a1[0].astype(jnp.float32)
        h2 = dv[...] * (u2f + ta[...].astype(jnp.float32))
        h2 = jnp.maximum(h2 + bias2[...], 0.0)
        rows = i * RB + lax.broadcasted_iota(jnp.int32, (RB, 1), 0)
        h2 = jnp.where(rows < N, h2, 0.0)
        t3 = dv[...] * jnp.dot(h2, w3[...], preferred_element_type=jnp.float32)
        t3_ref[...] = t3.astype(jnp.bfloat16)

    return pl.pallas_call(
        body,
        grid=(GRID,),
        in_specs=[
            _sp((1, RB, 64), lambda i: (0, i, 0)),
            _sp((1, RB, 64), lambda i: (1, i, 0)),
            _sp((RB, 64), lambda i: (i, 0)),
            _sp((RB, 1), lambda i: (i, 0)),
            _sp((1, 64), lambda i: (0, 0)),
            _sp((64, 32), lambda i: (0, 0)),
        ],
        out_specs=_sp((RB, 32), lambda i: (i, 0)),
        out_shape=jax.ShapeDtypeStruct((N_PAD, 32), jnp.bfloat16),
    )(u2, u2, t2, dinv, b2, W3)


def _tc_k4(u3, t3, dinv, b3, Wp1, bp1, Wp2, bp2):
    def body(ua, ub, tb, dv, bias3, wp1, biasp1, wp2, biasp2, out_ref, acc):
        i = pl.program_id(0)
        u3f = ua[0].astype(jnp.float32) + ub[0].astype(jnp.float32)
        h3 = dv[...] * (u3f + tb[...].astype(jnp.float32))
        h3 = jnp.maximum(h3 + bias3[...], 0.0)
        rows = i * RB + lax.broadcasted_iota(jnp.int32, (RB, 1), 0)
        h3 = jnp.where(rows < N, h3, 0.0)
        part = jnp.sum(h3, axis=0, keepdims=True)

        @pl.when(i == 0)
        def _():
            acc[...] = part

        @pl.when(i > 0)
        def _():
            acc[...] = acc[...] + part

        @pl.when(i == GRID - 1)
        def _():
            g = acc[...] * (1.0 / N)
            p = jnp.maximum(
                jnp.dot(g, wp1[...], preferred_element_type=jnp.float32)
                + biasp1[...], 0.0)
            out_ref[...] = (jnp.dot(p, wp2[...],
                                    preferred_element_type=jnp.float32)
                            + biasp2[...])

    return pl.pallas_call(
        body,
        grid=(GRID,),
        in_specs=[
            _sp((1, RB, 32), lambda i: (0, i, 0)),
            _sp((1, RB, 32), lambda i: (1, i, 0)),
            _sp((RB, 32), lambda i: (i, 0)),
            _sp((RB, 1), lambda i: (i, 0)),
            _sp((1, 32), lambda i: (0, 0)),
            _sp((32, 16), lambda i: (0, 0)),
            _sp((1, 16), lambda i: (0, 0)),
            _sp((16, 1), lambda i: (0, 0)),
            _sp((1, 1), lambda i: (0, 0)),
        ],
        out_specs=_sp((1, 1), lambda i: (0, 0)),
        out_shape=jax.ShapeDtypeStruct((1, 1), jnp.float32),
        scratch_shapes=[pltpu.VMEM((1, 32), jnp.float32)],
    )(u3, u3, t3, dinv, b3, Wp1, bp1, Wp2, bp2)


def kernel(x, edge_index, W1, b1, W2, b2, W3, b3, Wp1, bp1, Wp2, bp2):
    pad_cols = jnp.full((2, E_PAD - E), N, jnp.int32)
    ei = jnp.concatenate([edge_index, pad_cols], axis=1)
    src_hbm = ei[0].reshape(2, 16, N_CHUNKS, CHUNK)
    dst_hbm = ei[1].reshape(2, 16, N_CHUNKS, CHUNK)

    ones8 = jnp.ones((CHUNK, 8), jnp.float32)
    zeros8 = jnp.zeros((CHUNK, 8), jnp.float32)
    zeros64b = jnp.zeros((CHUNK, 64), jnp.bfloat16)
    zeros32b = jnp.zeros((CHUNK, 32), jnp.bfloat16)

    deg = _sc_degree_kernel()(dst_hbm, ones8, zeros8)
    dinv, t1 = _tc_k1(deg, x)

    u1 = _make_sc_propagate(8)(t1, src_hbm, dst_hbm, zeros8)
    t2 = _tc_k2(u1, t1, dinv, W1, b1.reshape(1, 128), W2)

    u2 = _make_sc_propagate(64, jnp.bfloat16)(t2, src_hbm, dst_hbm, zeros64b)
    t3 = _tc_k3(u2, t2, dinv, b2.reshape(1, 64), W3)

    u3 = _make_sc_propagate(32, jnp.bfloat16)(t3, src_hbm, dst_hbm, zeros32b)
    out = _tc_k4(u3, t3, dinv, b3.reshape(1, 32),
                 Wp1, bp1.reshape(1, 16), Wp2, bp2.reshape(1, 1))
    return out.reshape(1)


# speedup vs baseline: 1.5723x; 1.0095x over previous
import functools

import jax
import jax.numpy as jnp
from jax import lax
from jax.experimental import pallas as pl
from jax.experimental.pallas import tpu as pltpu
from jax.experimental.pallas import tpu_sc as plsc

N = 50000
E = 800000
D_IN = 8
N_PAD = 50176
E_PAD = 802816
N_CHUNKS = 196
CHUNK = 128
ROWS_PER_TILE = N_PAD // 16
RB = 3584
GRID = N_PAD // RB

_SC_PARAMS = pltpu.CompilerParams(use_tc_tiling_on_sc=False)


@functools.lru_cache(maxsize=None)
def _mesh():
    return plsc.VectorSubcoreMesh(core_axis_name="c", subcore_axis_name="s")


def _zero_acc(zeros_ref, zb_v, acc_sh, base):
    pltpu.sync_copy(zeros_ref, zb_v)

    def cp(k, carry):
        pltpu.sync_copy(zb_v, acc_sh.at[pl.ds(base + k * CHUNK, CHUNK)])
        return carry

    lax.fori_loop(0, ROWS_PER_TILE // CHUNK, cp, 0)
    rem = ROWS_PER_TILE - (ROWS_PER_TILE // CHUNK) * CHUNK
    if rem:
        pltpu.sync_copy(
            zb_v.at[pl.ds(0, rem)],
            acc_sh.at[pl.ds(base + (ROWS_PER_TILE // CHUNK) * CHUNK, rem)])


@functools.lru_cache(maxsize=None)
def _sc_degree_kernel():
    @functools.partial(
        pl.kernel,
        out_type=jax.ShapeDtypeStruct((2, N_PAD, 8), jnp.float32),
        mesh=_mesh(),
        scratch_types=[
            pltpu.VMEM((1, CHUNK), jnp.int32),
            pltpu.VMEM((1, CHUNK), jnp.int32),
            pltpu.VMEM((CHUNK, 8), jnp.float32),
            pltpu.VMEM((CHUNK, 8), jnp.float32),
            pltpu.SemaphoreType.DMA,
            pltpu.SemaphoreType.DMA,
            pltpu.VMEM_SHARED((N_PAD, 8), jnp.float32),
        ],
        compiler_params=_SC_PARAMS,
    )
    def deg_kernel(dst_ref, ones_ref, zeros_ref, out_ref, dst_a, dst_b,
                   ones_v, zb_v, sem_a, sem_b, acc_sh):
        c = lax.axis_index("c")
        s = lax.axis_index("s")
        base = s * ROWS_PER_TILE
        _zero_acc(zeros_ref, zb_v, acc_sh, base)
        pltpu.sync_copy(ones_ref, ones_v)
        plsc.subcore_barrier()

        pltpu.async_copy(dst_ref.at[c, s, 0], dst_a.at[0], sem_a)
        pltpu.async_copy(dst_ref.at[c, s, 1], dst_b.at[0], sem_b)

        def body(i, carry):
            j0 = 2 * i
            pltpu.make_async_copy(dst_ref.at[c, s, j0], dst_a.at[0],
                                  sem_a).wait()
            pltpu.sync_copy(ones_v, acc_sh.at[dst_a.at[0]], add=True)

            @pl.when(j0 + 2 < N_CHUNKS)
            def _():
                pltpu.async_copy(dst_ref.at[c, s, j0 + 2], dst_a.at[0], sem_a)

            pltpu.make_async_copy(dst_ref.at[c, s, j0 + 1], dst_b.at[0],
                                  sem_b).wait()
            pltpu.sync_copy(ones_v, acc_sh.at[dst_b.at[0]], add=True)

            @pl.when(j0 + 3 < N_CHUNKS)
            def _():
                pltpu.async_copy(dst_ref.at[c, s, j0 + 3], dst_b.at[0], sem_b)

            return carry

        lax.fori_loop(0, N_CHUNKS // 2, body, 0)
        plsc.subcore_barrier()
        pltpu.sync_copy(acc_sh.at[pl.ds(base, ROWS_PER_TILE)],
                        out_ref.at[c, pl.ds(base, ROWS_PER_TILE)])

    return deg_kernel


@functools.lru_cache(maxsize=None)
def _make_sc_propagate(D, dtype=jnp.float32):
    @functools.partial(
        pl.kernel,
        out_type=jax.ShapeDtypeStruct((2, N_PAD, D), dtype),
        mesh=_mesh(),
        scratch_types=[
            pltpu.VMEM((1, CHUNK), jnp.int32),
            pltpu.VMEM((1, CHUNK), jnp.int32),
            pltpu.VMEM((1, CHUNK), jnp.int32),
            pltpu.VMEM((1, CHUNK), jnp.int32),
            pltpu.VMEM((CHUNK, D), dtype),
            pltpu.VMEM((CHUNK, D), dtype),
            pltpu.SemaphoreType.DMA,
            pltpu.SemaphoreType.DMA,
            pltpu.SemaphoreType.DMA,
            pltpu.SemaphoreType.DMA,
            pltpu.SemaphoreType.DMA,
            pltpu.SemaphoreType.DMA,
            pltpu.VMEM_SHARED((N_PAD, D), dtype),
        ],
        compiler_params=_SC_PARAMS,
    )
    def prop_kernel(table_ref, src_ref, dst_ref, zeros_ref, out_ref,
                    src_a, dst_a, src_b, dst_b, rows_a, rows_b,
                    sem_sa, sem_da, sem_sb, sem_db, sem_ga, sem_gb, acc_sh):
        c = lax.axis_index("c")
        s = lax.axis_index("s")
        base = s * ROWS_PER_TILE
        _zero_acc(zeros_ref, rows_a, acc_sh, base)
        plsc.subcore_barrier()

        pltpu.async_copy(src_ref.at[c, s, 0], src_a.at[0], sem_sa)
        pltpu.async_copy(dst_ref.at[c, s, 0], dst_a.at[0], sem_da)
        pltpu.async_copy(src_ref.at[c, s, 1], src_b.at[0], sem_sb)
        pltpu.async_copy(dst_ref.at[c, s, 1], dst_b.at[0], sem_db)

        def body(i, carry):
            j0 = 2 * i
            pltpu.make_async_copy(src_ref.at[c, s, j0], src_a.at[0],
                                  sem_sa).wait()
            cp_a = pltpu.async_copy(table_ref.at[src_a.at[0]], rows_a, sem_ga)
            pltpu.make_async_copy(src_ref.at[c, s, j0 + 1], src_b.at[0],
                                  sem_sb).wait()
            cp_b = pltpu.async_copy(table_ref.at[src_b.at[0]], rows_b, sem_gb)

            pltpu.make_async_copy(dst_ref.at[c, s, j0], dst_a.at[0],
                                  sem_da).wait()
            cp_a.wait()
            pltpu.sync_copy(rows_a, acc_sh.at[dst_a.at[0]], add=True)

            @pl.when(j0 + 2 < N_CHUNKS)
            def _():
                pltpu.async_copy(src_ref.at[c, s, j0 + 2], src_a.at[0], sem_sa)
                pltpu.async_copy(dst_ref.at[c, s, j0 + 2], dst_a.at[0], sem_da)

            pltpu.make_async_copy(dst_ref.at[c, s, j0 + 1], dst_b.at[0],
                                  sem_db).wait()
            cp_b.wait()
            pltpu.sync_copy(rows_b, acc_sh.at[dst_b.at[0]], add=True)

            @pl.when(j0 + 3 < N_CHUNKS)
            def _():
                pltpu.async_copy(src_ref.at[c, s, j0 + 3], src_b.at[0], sem_sb)
                pltpu.async_copy(dst_ref.at[c, s, j0 + 3], dst_b.at[0], sem_db)

            return carry

        lax.fori_loop(0, N_CHUNKS // 2, body, 0)
        plsc.subcore_barrier()
        pltpu.sync_copy(acc_sh.at[pl.ds(base, ROWS_PER_TILE)],
                        out_ref.at[c, pl.ds(base, ROWS_PER_TILE)])

    return prop_kernel


def _sp(shape, imap):
    return pl.BlockSpec(shape, imap)


def _tc_k1(deg3, x):
    def body(dA, dB, xb, dinv_ref, t1_ref):
        i = pl.program_id(0)
        deg = (dA[0][:, 0:1].astype(jnp.float32)
               + dB[0][:, 0:1].astype(jnp.float32) + 1.0)
        dinv = lax.rsqrt(deg)
        rows = i * RB + lax.broadcasted_iota(jnp.int32, (RB, 1), 0)
        dinv_ref[...] = dinv.astype(jnp.bfloat16)
        t1_ref[...] = jnp.where(rows < N, xb[...] * dinv, 0.0)

    return pl.pallas_call(
        body,
        grid=(GRID,),
        in_specs=[
            _sp((1, RB, 8), lambda i: (0, i, 0)),
            _sp((1, RB, 8), lambda i: (1, i, 0)),
            _sp((RB, 8), lambda i: (i, 0)),
        ],
        out_specs=[
            _sp((RB, 1), lambda i: (i, 0)),
            _sp((RB, 8), lambda i: (i, 0)),
        ],
        out_shape=[
            jax.ShapeDtypeStruct((N_PAD, 1), jnp.bfloat16),
            jax.ShapeDtypeStruct((N_PAD, 8), jnp.float32),
        ],
    )(deg3, deg3, x)


def _tc_k2(u1, t1, dinv, W1, b1, W2):
    def body(ua, ub, t1b, dv, w1, bias1, w2, t2_ref):
        i = pl.program_id(0)
        dvf = dv[...].astype(jnp.float32)
        sh = dvf * (ua[0].astype(jnp.float32) + ub[0].astype(jnp.float32)
                    + t1b[...].astype(jnp.float32))
        h1 = jnp.maximum(
            jnp.dot(sh, w1[...], preferred_element_type=jnp.float32) + bias1[...],
            0.0)
        rows = i * RB + lax.broadcasted_iota(jnp.int32, (RB, 1), 0)
        h1 = jnp.where(rows < N, h1, 0.0)
        t2 = dvf * jnp.dot(h1, w2[...], preferred_element_type=jnp.float32)
        t2_ref[...] = t2.astype(jnp.bfloat16)

    return pl.pallas_call(
        body,
        grid=(GRID,),
        in_specs=[
            _sp((1, RB, 8), lambda i: (0, i, 0)),
            _sp((1, RB, 8), lambda i: (1, i, 0)),
            _sp((RB, 8), lambda i: (i, 0)),
            _sp((RB, 1), lambda i: (i, 0)),
            _sp((8, 128), lambda i: (0, 0)),
            _sp((1, 128), lambda i: (0, 0)),
            _sp((128, 64), lambda i: (0, 0)),
        ],
        out_specs=_sp((RB, 64), lambda i: (i, 0)),
        out_shape=jax.ShapeDtypeStruct((N_PAD, 64), jnp.bfloat16),
    )(u1, u1, t1, dinv, W1, b1, W2)


def _tc_k3(u2, t2, dinv, b2, W3):
    def body(a0, a1, ta, dv, bias2, w3, t3_ref):
        i = pl.program_id(0)
        u2f = a0[0].astype(jnp.float32) + a1[0].astype(jnp.float32)
        dvf = dv[...].astype(jnp.float32)
        h2 = dvf * (u2f + ta[...].astype(jnp.float32))
        h2 = jnp.maximum(h2 + bias2[...], 0.0)
        rows = i * RB + lax.broadcasted_iota(jnp.int32, (RB, 1), 0)
        h2 = jnp.where(rows < N, h2, 0.0)
        t3 = dvf * jnp.dot(h2, w3[...], preferred_element_type=jnp.float32)
        t3_ref[...] = t3.astype(jnp.bfloat16)

    return pl.pallas_call(
        body,
        grid=(GRID,),
        in_specs=[
            _sp((1, RB, 64), lambda i: (0, i, 0)),
            _sp((1, RB, 64), lambda i: (1, i, 0)),
            _sp((RB, 64), lambda i: (i, 0)),
            _sp((RB, 1), lambda i: (i, 0)),
            _sp((1, 64), lambda i: (0, 0)),
            _sp((64, 32), lambda i: (0, 0)),
        ],
        out_specs=_sp((RB, 32), lambda i: (i, 0)),
        out_shape=jax.ShapeDtypeStruct((N_PAD, 32), jnp.bfloat16),
    )(u2, u2, t2, dinv, b2, W3)


def _tc_k4(u3, t3, dinv, b3, Wp1, bp1, Wp2, bp2):
    def body(ua, ub, tb, dv, bias3, wp1, biasp1, wp2, biasp2, out_ref, acc):
        i = pl.program_id(0)
        u3f = ua[0].astype(jnp.float32) + ub[0].astype(jnp.float32)
        h3 = dv[...].astype(jnp.float32) * (u3f + tb[...].astype(jnp.float32))
        h3 = jnp.maximum(h3 + bias3[...], 0.0)
        rows = i * RB + lax.broadcasted_iota(jnp.int32, (RB, 1), 0)
        h3 = jnp.where(rows < N, h3, 0.0)
        part = jnp.sum(h3, axis=0, keepdims=True)

        @pl.when(i == 0)
        def _():
            acc[...] = part

        @pl.when(i > 0)
        def _():
            acc[...] = acc[...] + part

        @pl.when(i == GRID - 1)
        def _():
            g = acc[...] * (1.0 / N)
            p = jnp.maximum(
                jnp.dot(g, wp1[...], preferred_element_type=jnp.float32)
                + biasp1[...], 0.0)
            out_ref[...] = (jnp.dot(p, wp2[...],
                                    preferred_element_type=jnp.float32)
                            + biasp2[...])

    return pl.pallas_call(
        body,
        grid=(GRID,),
        in_specs=[
            _sp((1, RB, 32), lambda i: (0, i, 0)),
            _sp((1, RB, 32), lambda i: (1, i, 0)),
            _sp((RB, 32), lambda i: (i, 0)),
            _sp((RB, 1), lambda i: (i, 0)),
            _sp((1, 32), lambda i: (0, 0)),
            _sp((32, 16), lambda i: (0, 0)),
            _sp((1, 16), lambda i: (0, 0)),
            _sp((16, 1), lambda i: (0, 0)),
            _sp((1, 1), lambda i: (0, 0)),
        ],
        out_specs=_sp((1, 1), lambda i: (0, 0)),
        out_shape=jax.ShapeDtypeStruct((1, 1), jnp.float32),
        scratch_shapes=[pltpu.VMEM((1, 32), jnp.float32)],
    )(u3, u3, t3, dinv, b3, Wp1, bp1, Wp2, bp2)


def kernel(x, edge_index, W1, b1, W2, b2, W3, b3, Wp1, bp1, Wp2, bp2):
    pad_cols = jnp.full((2, E_PAD - E), N, jnp.int32)
    ei = jnp.concatenate([edge_index, pad_cols], axis=1)
    src_hbm = ei[0].reshape(2, 16, N_CHUNKS, CHUNK)
    dst_hbm = ei[1].reshape(2, 16, N_CHUNKS, CHUNK)

    ones8 = jnp.ones((CHUNK, 8), jnp.float32)
    zeros8 = jnp.zeros((CHUNK, 8), jnp.float32)
    zeros64b = jnp.zeros((CHUNK, 64), jnp.bfloat16)
    zeros32b = jnp.zeros((CHUNK, 32), jnp.bfloat16)

    deg = _sc_degree_kernel()(dst_hbm, ones8, zeros8)
    dinv, t1 = _tc_k1(deg, x)

    u1 = _make_sc_propagate(8)(t1, src_hbm, dst_hbm, zeros8)
    t2 = _tc_k2(u1, t1, dinv, W1, b1.reshape(1, 128), W2)

    u2 = _make_sc_propagate(64, jnp.bfloat16)(t2, src_hbm, dst_hbm, zeros64b)
    t3 = _tc_k3(u2, t2, dinv, b2.reshape(1, 64), W3)

    u3 = _make_sc_propagate(32, jnp.bfloat16)(t3, src_hbm, dst_hbm, zeros32b)
    out = _tc_k4(u3, t3, dinv, b3.reshape(1, 32),
                 Wp1, bp1.reshape(1, 16), Wp2, bp2.reshape(1, 1))
    return out.reshape(1)


# speedup vs baseline: 1.6994x; 1.0808x over previous
import functools

import jax
import jax.numpy as jnp
from jax import lax
from jax.experimental import pallas as pl
from jax.experimental.pallas import tpu as pltpu
from jax.experimental.pallas import tpu_sc as plsc

N = 50000
E = 800000
D_IN = 8
N_PAD = 50176
E_PAD = 802816
N_CHUNKS = 196
CHUNK = 128
ROWS_PER_TILE = N_PAD // 16
RB = 3584
GRID = N_PAD // RB

_SC_PARAMS = pltpu.CompilerParams(use_tc_tiling_on_sc=False)


@functools.lru_cache(maxsize=None)
def _mesh():
    return plsc.VectorSubcoreMesh(core_axis_name="c", subcore_axis_name="s")


def _zero_acc(zeros_ref, zb_v, acc_sh, base):
    pltpu.sync_copy(zeros_ref, zb_v)

    def cp(k, carry):
        pltpu.sync_copy(zb_v, acc_sh.at[pl.ds(base + k * CHUNK, CHUNK)])
        return carry

    lax.fori_loop(0, ROWS_PER_TILE // CHUNK, cp, 0)
    rem = ROWS_PER_TILE - (ROWS_PER_TILE // CHUNK) * CHUNK
    if rem:
        pltpu.sync_copy(
            zb_v.at[pl.ds(0, rem)],
            acc_sh.at[pl.ds(base + (ROWS_PER_TILE // CHUNK) * CHUNK, rem)])


@functools.lru_cache(maxsize=None)
def _sc_degree_kernel():
    @functools.partial(
        pl.kernel,
        out_type=jax.ShapeDtypeStruct((2, N_PAD, 8), jnp.float32),
        mesh=_mesh(),
        scratch_types=[
            pltpu.VMEM((1, CHUNK), jnp.int32),
            pltpu.VMEM((1, CHUNK), jnp.int32),
            pltpu.VMEM((CHUNK, 8), jnp.float32),
            pltpu.VMEM((CHUNK, 8), jnp.float32),
            pltpu.SemaphoreType.DMA,
            pltpu.SemaphoreType.DMA,
            pltpu.VMEM_SHARED((N_PAD, 8), jnp.float32),
        ],
        compiler_params=_SC_PARAMS,
    )
    def deg_kernel(dst_ref, ones_ref, zeros_ref, out_ref, dst_a, dst_b,
                   ones_v, zb_v, sem_a, sem_b, acc_sh):
        c = lax.axis_index("c")
        s = lax.axis_index("s")
        base = s * ROWS_PER_TILE
        _zero_acc(zeros_ref, zb_v, acc_sh, base)
        pltpu.sync_copy(ones_ref, ones_v)
        plsc.subcore_barrier()

        pltpu.async_copy(dst_ref.at[c, s, 0], dst_a.at[0], sem_a)
        pltpu.async_copy(dst_ref.at[c, s, 1], dst_b.at[0], sem_b)

        def body(i, carry):
            j0 = 2 * i
            pltpu.make_async_copy(dst_ref.at[c, s, j0], dst_a.at[0],
                                  sem_a).wait()
            pltpu.sync_copy(ones_v, acc_sh.at[dst_a.at[0]], add=True)

            @pl.when(j0 + 2 < N_CHUNKS)
            def _():
                pltpu.async_copy(dst_ref.at[c, s, j0 + 2], dst_a.at[0], sem_a)

            pltpu.make_async_copy(dst_ref.at[c, s, j0 + 1], dst_b.at[0],
                                  sem_b).wait()
            pltpu.sync_copy(ones_v, acc_sh.at[dst_b.at[0]], add=True)

            @pl.when(j0 + 3 < N_CHUNKS)
            def _():
                pltpu.async_copy(dst_ref.at[c, s, j0 + 3], dst_b.at[0], sem_b)

            return carry

        lax.fori_loop(0, N_CHUNKS // 2, body, 0)
        plsc.subcore_barrier()
        pltpu.sync_copy(acc_sh.at[pl.ds(base, ROWS_PER_TILE)],
                        out_ref.at[c, pl.ds(base, ROWS_PER_TILE)])

    return deg_kernel


@functools.lru_cache(maxsize=None)
def _make_sc_propagate(D, dtype=jnp.float32):
    NBLK = 7
    BCH = N_CHUNKS // NBLK
    assert BCH % 2 == 0 and NBLK * BCH == N_CHUNKS

    @functools.partial(
        pl.kernel,
        out_type=jax.ShapeDtypeStruct((2, N_PAD, D), dtype),
        mesh=_mesh(),
        scratch_types=[
            pltpu.VMEM((BCH, CHUNK), jnp.int32),
            pltpu.VMEM((BCH, CHUNK), jnp.int32),
            pltpu.VMEM((CHUNK, D), dtype),
            pltpu.VMEM((CHUNK, D), dtype),
            pltpu.SemaphoreType.DMA,
            pltpu.SemaphoreType.DMA,
            pltpu.SemaphoreType.DMA,
            pltpu.SemaphoreType.DMA,
            pltpu.VMEM_SHARED((N_PAD, D), dtype),
        ],
        compiler_params=_SC_PARAMS,
    )
    def prop_kernel(table_ref, src_ref, dst_ref, zeros_ref, out_ref,
                    sidx, didx, rows_a, rows_b,
                    sem_ga, sem_gb, sem_sca, sem_scb, acc_sh):
        c = lax.axis_index("c")
        s = lax.axis_index("s")
        base = s * ROWS_PER_TILE
        _zero_acc(zeros_ref, rows_a, acc_sh, base)
        plsc.subcore_barrier()

        def blk_body(blk, carry):
            @pl.when(blk >= 1)
            def _():
                pltpu.make_async_copy(rows_a, acc_sh.at[didx.at[0]],
                                      sem_sca).wait()
                pltpu.make_async_copy(rows_b, acc_sh.at[didx.at[0]],
                                      sem_scb).wait()

            j0 = blk * BCH
            pltpu.sync_copy(src_ref.at[c, s, pl.ds(j0, BCH)], sidx)
            pltpu.sync_copy(dst_ref.at[c, s, pl.ds(j0, BCH)], didx)

            def body(i, carry2):
                @pl.when(i >= 1)
                def _():
                    pltpu.make_async_copy(rows_a, acc_sh.at[didx.at[0]],
                                          sem_sca).wait()
                ca = pltpu.async_copy(table_ref.at[sidx.at[2 * i]], rows_a,
                                      sem_ga)

                @pl.when(i >= 1)
                def _():
                    pltpu.make_async_copy(rows_b, acc_sh.at[didx.at[0]],
                                          sem_scb).wait()
                cb = pltpu.async_copy(table_ref.at[sidx.at[2 * i + 1]], rows_b,
                                      sem_gb)

                ca.wait()
                pltpu.async_copy(rows_a, acc_sh.at[didx.at[2 * i]], sem_sca,
                                 add=True)
                cb.wait()
                pltpu.async_copy(rows_b, acc_sh.at[didx.at[2 * i + 1]], sem_scb,
                                 add=True)
                return carry2

            lax.fori_loop(0, BCH // 2, body, 0)
            return carry

        lax.fori_loop(0, NBLK, blk_body, 0)
        pltpu.make_async_copy(rows_a, acc_sh.at[didx.at[0]], sem_sca).wait()
        pltpu.make_async_copy(rows_b, acc_sh.at[didx.at[0]], sem_scb).wait()
        plsc.subcore_barrier()
        pltpu.sync_copy(acc_sh.at[pl.ds(base, ROWS_PER_TILE)],
                        out_ref.at[c, pl.ds(base, ROWS_PER_TILE)])

    return prop_kernel


def _sp(shape, imap):
    return pl.BlockSpec(shape, imap)


def _tc_k1(deg3, x):
    def body(dA, dB, xb, dinv_ref, t1_ref):
        i = pl.program_id(0)
        deg = (dA[0][:, 0:1].astype(jnp.float32)
               + dB[0][:, 0:1].astype(jnp.float32) + 1.0)
        dinv = lax.rsqrt(deg)
        rows = i * RB + lax.broadcasted_iota(jnp.int32, (RB, 1), 0)
        dinv_ref[...] = dinv.astype(jnp.bfloat16)
        t1_ref[...] = jnp.where(rows < N, xb[...] * dinv, 0.0)

    return pl.pallas_call(
        body,
        grid=(GRID,),
        in_specs=[
            _sp((1, RB, 8), lambda i: (0, i, 0)),
            _sp((1, RB, 8), lambda i: (1, i, 0)),
            _sp((RB, 8), lambda i: (i, 0)),
        ],
        out_specs=[
            _sp((RB, 1), lambda i: (i, 0)),
            _sp((RB, 8), lambda i: (i, 0)),
        ],
        out_shape=[
            jax.ShapeDtypeStruct((N_PAD, 1), jnp.bfloat16),
            jax.ShapeDtypeStruct((N_PAD, 8), jnp.float32),
        ],
    )(deg3, deg3, x)


def _tc_k2(u1, t1, dinv, W1, b1, W2):
    def body(ua, ub, t1b, dv, w1, bias1, w2, t2_ref):
        i = pl.program_id(0)
        dvf = dv[...].astype(jnp.float32)
        sh = dvf * (ua[0].astype(jnp.float32) + ub[0].astype(jnp.float32)
                    + t1b[...].astype(jnp.float32))
        h1 = jnp.maximum(
            jnp.dot(sh, w1[...], preferred_element_type=jnp.float32) + bias1[...],
            0.0)
        rows = i * RB + lax.broadcasted_iota(jnp.int32, (RB, 1), 0)
        h1 = jnp.where(rows < N, h1, 0.0)
        t2 = dvf * jnp.dot(h1, w2[...], preferred_element_type=jnp.float32)
        t2_ref[...] = t2.astype(jnp.bfloat16)

    return pl.pallas_call(
        body,
        grid=(GRID,),
        in_specs=[
            _sp((1, RB, 8), lambda i: (0, i, 0)),
            _sp((1, RB, 8), lambda i: (1, i, 0)),
            _sp((RB, 8), lambda i: (i, 0)),
            _sp((RB, 1), lambda i: (i, 0)),
            _sp((8, 128), lambda i: (0, 0)),
            _sp((1, 128), lambda i: (0, 0)),
            _sp((128, 64), lambda i: (0, 0)),
        ],
        out_specs=_sp((RB, 64), lambda i: (i, 0)),
        out_shape=jax.ShapeDtypeStruct((N_PAD, 64), jnp.bfloat16),
    )(u1, u1, t1, dinv, W1, b1, W2)


def _tc_k3(u2, t2, dinv, b2, W3):
    def body(a0, a1, ta, dv, bias2, w3, t3_ref):
        i = pl.program_id(0)
        u2f = a0[0].astype(jnp.float32) + a1[0].astype(jnp.float32)
        dvf = dv[...].astype(jnp.float32)
        h2 = dvf * (u2f + ta[...].astype(jnp.float32))
        h2 = jnp.maximum(h2 + bias2[...], 0.0)
        rows = i * RB + lax.broadcasted_iota(jnp.int32, (RB, 1), 0)
        h2 = jnp.where(rows < N, h2, 0.0)
        t3 = dvf * jnp.dot(h2, w3[...], preferred_element_type=jnp.float32)
        t3_ref[...] = t3.astype(jnp.bfloat16)

    return pl.pallas_call(
        body,
        grid=(GRID,),
        in_specs=[
            _sp((1, RB, 64), lambda i: (0, i, 0)),
            _sp((1, RB, 64), lambda i: (1, i, 0)),
            _sp((RB, 64), lambda i: (i, 0)),
            _sp((RB, 1), lambda i: (i, 0)),
            _sp((1, 64), lambda i: (0, 0)),
            _sp((64, 32), lambda i: (0, 0)),
        ],
        out_specs=_sp((RB, 32), lambda i: (i, 0)),
        out_shape=jax.ShapeDtypeStruct((N_PAD, 32), jnp.bfloat16),
    )(u2, u2, t2, dinv, b2, W3)


def _tc_k4(u3, t3, dinv, b3, Wp1, bp1, Wp2, bp2):
    def body(ua, ub, tb, dv, bias3, wp1, biasp1, wp2, biasp2, out_ref, acc):
        i = pl.program_id(0)
        u3f = ua[0].astype(jnp.float32) + ub[0].astype(jnp.float32)
        h3 = dv[...].astype(jnp.float32) * (u3f + tb[...].astype(jnp.float32))
        h3 = jnp.maximum(h3 + bias3[...], 0.0)
        rows = i * RB + lax.broadcasted_iota(jnp.int32, (RB, 1), 0)
        h3 = jnp.where(rows < N, h3, 0.0)
        part = jnp.sum(h3, axis=0, keepdims=True)

        @pl.when(i == 0)
        def _():
            acc[...] = part

        @pl.when(i > 0)
        def _():
            acc[...] = acc[...] + part

        @pl.when(i == GRID - 1)
        def _():
            g = acc[...] * (1.0 / N)
            p = jnp.maximum(
                jnp.dot(g, wp1[...], preferred_element_type=jnp.float32)
                + biasp1[...], 0.0)
            out_ref[...] = (jnp.dot(p, wp2[...],
                                    preferred_element_type=jnp.float32)
                            + biasp2[...])

    return pl.pallas_call(
        body,
        grid=(GRID,),
        in_specs=[
            _sp((1, RB, 32), lambda i: (0, i, 0)),
            _sp((1, RB, 32), lambda i: (1, i, 0)),
            _sp((RB, 32), lambda i: (i, 0)),
            _sp((RB, 1), lambda i: (i, 0)),
            _sp((1, 32), lambda i: (0, 0)),
            _sp((32, 16), lambda i: (0, 0)),
            _sp((1, 16), lambda i: (0, 0)),
            _sp((16, 1), lambda i: (0, 0)),
            _sp((1, 1), lambda i: (0, 0)),
        ],
        out_specs=_sp((1, 1), lambda i: (0, 0)),
        out_shape=jax.ShapeDtypeStruct((1, 1), jnp.float32),
        scratch_shapes=[pltpu.VMEM((1, 32), jnp.float32)],
    )(u3, u3, t3, dinv, b3, Wp1, bp1, Wp2, bp2)


def kernel(x, edge_index, W1, b1, W2, b2, W3, b3, Wp1, bp1, Wp2, bp2):
    pad_cols = jnp.full((2, E_PAD - E), N, jnp.int32)
    ei = jnp.concatenate([edge_index, pad_cols], axis=1)
    src_hbm = ei[0].reshape(2, 16, N_CHUNKS, CHUNK)
    dst_hbm = ei[1].reshape(2, 16, N_CHUNKS, CHUNK)

    ones8 = jnp.ones((CHUNK, 8), jnp.float32)
    zeros8 = jnp.zeros((CHUNK, 8), jnp.float32)
    zeros64b = jnp.zeros((CHUNK, 64), jnp.bfloat16)
    zeros32b = jnp.zeros((CHUNK, 32), jnp.bfloat16)

    deg = _sc_degree_kernel()(dst_hbm, ones8, zeros8)
    dinv, t1 = _tc_k1(deg, x)

    u1 = _make_sc_propagate(8)(t1, src_hbm, dst_hbm, zeros8)
    t2 = _tc_k2(u1, t1, dinv, W1, b1.reshape(1, 128), W2)

    u2 = _make_sc_propagate(64, jnp.bfloat16)(t2, src_hbm, dst_hbm, zeros64b)
    t3 = _tc_k3(u2, t2, dinv, b2.reshape(1, 64), W3)

    u3 = _make_sc_propagate(32, jnp.bfloat16)(t3, src_hbm, dst_hbm, zeros32b)
    out = _tc_k4(u3, t3, dinv, b3.reshape(1, 32),
                 Wp1, bp1.reshape(1, 16), Wp2, bp2.reshape(1, 1))
    return out.reshape(1)
